# baseline (speedup 1.0000x reference)
import functools

import jax
import jax.numpy as jnp
from jax import lax
from jax.experimental import pallas as pl
from jax.experimental.pallas import tpu as pltpu
from jax.experimental.pallas import tpu_sc as plsc

_info = plsc.get_sparse_core_info()
_NC, _NS = _info.num_cores, _info.num_subcores
_NW = _NC * _NS

_VOCAB = 1000000
_D = 32
_B_TOTAL = 4096 * 200
_B_PER_W = _B_TOTAL // _NW
_CH = 640
_NCH = _B_PER_W // _CH
_NBUF = 4

_CHV = 512
_NCK = 1953
_RM_ROWS = _VOCAB * _D // 128
_RM_PER_CK = _CHV * _D // 128

_mesh = plsc.VectorSubcoreMesh(core_axis_name="c", subcore_axis_name="s")


@functools.partial(
    pl.kernel,
    out_type=jax.ShapeDtypeStruct((_RM_ROWS, 128), jnp.float32),
    mesh=_mesh,
    scratch_types=[
        [pltpu.VMEM((_D, _CHV + 1), jnp.float32) for _ in range(2)],
        [pltpu.VMEM((_RM_PER_CK, 128), jnp.float32) for _ in range(2)],
        pltpu.VMEM((_D, 64), jnp.float32),
        [pltpu.SemaphoreType.DMA for _ in range(2)],
        [pltpu.SemaphoreType.DMA for _ in range(2)],
    ],
    compiler_params=pltpu.CompilerParams(
        use_tc_tiling_on_sc=True, needs_layout_passes=False),
)
def _transpose_kernel(tT_hbm, rm_hbm, inb, outb, tailb, gsem, wsem):
    wid = lax.axis_index("s") * _NC + lax.axis_index("c")
    nblk = _NCK // _NW + jnp.where(wid < _NCK % _NW, 1, 0)

    d_lo = lax.iota(jnp.int32, 16)
    d_hi = d_lo + 16

    def load_blk(vt, p):
        off = pl.multiple_of(vt * _CHV, 128)
        return pltpu.async_copy(
            tT_hbm.at[:, pl.ds(off, _CHV)],
            inb[p].at[:, pl.ds(0, _CHV)], gsem[p])

    def wait_load(p):
        pltpu.make_async_copy(
            tT_hbm.at[:, pl.ds(0, _CHV)],
            inb[p].at[:, pl.ds(0, _CHV)], gsem[p]).wait()

    def store_blk(r0, p):
        return pltpu.async_copy(
            outb[p],
            rm_hbm.at[pl.ds(pl.multiple_of(r0, _RM_PER_CK), _RM_PER_CK)],
            wsem[p])

    def wait_store(p):
        pltpu.make_async_copy(
            outb[p], rm_hbm.at[pl.ds(0, _RM_PER_CK)], wsem[p]).wait()

    def permute(p):
        @pl.loop(0, _RM_PER_CK, unroll=8)
        def _(j):
            for h in range(8):
                c = jnp.full((16,), 4 * j + h // 2, jnp.int32)
                d = d_lo if h % 2 == 0 else d_hi
                v = plsc.load_gather(inb[p], [d, c])
                outb[p][j, pl.ds(16 * h, 16)] = v

    def blk_index(i):
        return i * _NW + wid

    load_blk(blk_index(0), 0)

    @pl.loop(0, nblk)
    def _(i):
        p = (i % 2).astype(jnp.int32)
        for par in range(2):
            @pl.when(p == par)
            def _():
                @pl.when(i + 1 < nblk)
                def _():
                    load_blk(blk_index(i + 1), 1 - par)
                wait_load(par)
                @pl.when(i >= 2)
                def _():
                    wait_store(par)
                permute(par)
                store_blk(blk_index(i) * _RM_PER_CK, par)

    @pl.loop(0, jnp.minimum(nblk, 2))
    def _(i):
        for par in range(2):
            @pl.when(((nblk - 1 - i) % 2) == par)
            def _():
                wait_store(par)

    @pl.when(wid == _NW - 1)
    def _():
        pltpu.async_copy(
            tT_hbm.at[:, pl.ds(_NCK * _CHV, 64)], tailb, gsem[0]).wait()
        @pl.loop(0, 16)
        def _(j):
            for h in range(8):
                c = jnp.full((16,), 4 * j + h // 2, jnp.int32)
                d = d_lo if h % 2 == 0 else d_hi
                v = plsc.load_gather(tailb, [d, c])
                outb[0][j, pl.ds(16 * h, 16)] = v
        pltpu.async_copy(
            outb[0].at[pl.ds(0, 16)],
            rm_hbm.at[pl.ds(_RM_ROWS - 16, 16)], wsem[0]).wait()


@functools.partial(
    pl.kernel,
    out_type=jax.ShapeDtypeStruct((_B_TOTAL, _D), jnp.float32),
    mesh=_mesh,
    scratch_types=[
        pltpu.VMEM((_B_PER_W,), jnp.int32),
        [pltpu.VMEM((_CH, _D), jnp.float32) for _ in range(_NBUF)],
        [pltpu.SemaphoreType.DMA for _ in range(_NBUF)],
        [pltpu.SemaphoreType.DMA for _ in range(_NBUF)],
    ],
    compiler_params=pltpu.CompilerParams(use_tc_tiling_on_sc=False),
)
def _gather_kernel(idx_hbm, table_hbm, out_hbm, idx_v, rows, gsem, wsem):
    wid = lax.axis_index("s") * _NC + lax.axis_index("c")
    base = wid * _B_PER_W

    pltpu.sync_copy(idx_hbm.at[pl.ds(base, _B_PER_W)], idx_v)

    def gather(c, b):
        return pltpu.async_copy(
            table_hbm.at[idx_v.at[pl.ds(c * _CH, _CH)]], rows[b], gsem[b])

    def gather_wait(b):
        pltpu.make_async_copy(
            table_hbm.at[idx_v.at[pl.ds(0, _CH)]], rows[b], gsem[b]).wait()

    def write(c, b):
        return pltpu.async_copy(
            rows[b], out_hbm.at[pl.ds(base + c * _CH, _CH)], wsem[b])

    def write_wait(b):
        pltpu.make_async_copy(
            rows[b], out_hbm.at[pl.ds(base, _CH)], wsem[b]).wait()

    for b in range(_NBUF):
        gather(b, b)

    @pl.loop(0, _NCH, step=_NBUF)
    def _(c):
        for b in range(_NBUF):
            g = c + b
            gather_wait(b)
            write(g, b)

            @pl.when(g + _NBUF < _NCH)
            def _():
                write_wait(b)
                gather(g + _NBUF, b)

    for b in range(_NBUF):
        write_wait(b)


def kernel(x, table):
    table_rm = _transpose_kernel(table.T)
    out = _gather_kernel(x.reshape(-1), table_rm.reshape(_VOCAB, _D))
    return out.reshape(x.shape[0], x.shape[1], _D)


# speedup vs baseline: 1.3894x; 1.3894x over previous
import functools

import jax
import jax.numpy as jnp
from jax import lax
from jax.experimental import pallas as pl
from jax.experimental.pallas import tpu as pltpu
from jax.experimental.pallas import tpu_sc as plsc

_info = plsc.get_sparse_core_info()
_NC, _NS = _info.num_cores, _info.num_subcores
_NW = _NC * _NS

_VOCAB = 1000000
_D = 32
_B_TOTAL = 4096 * 200
_B_PER_W = _B_TOTAL // _NW
_CH = 640
_NCH = _B_PER_W // _CH
_NBUF = 4

_CHV = 512
_NCK = 1953
_RM_ROWS = _VOCAB * _D // 128
_RM_PER_CK = _CHV * _D // 128

_mesh = plsc.VectorSubcoreMesh(core_axis_name="c", subcore_axis_name="s")


@functools.partial(
    pl.kernel,
    out_type=jax.ShapeDtypeStruct((_RM_ROWS, 128), jnp.float32),
    mesh=_mesh,
    scratch_types=[
        [pltpu.VMEM((_D, _CHV + 1), jnp.float32) for _ in range(2)],
        [pltpu.VMEM((_RM_PER_CK, 128), jnp.float32) for _ in range(2)],
        pltpu.VMEM((_D, 64), jnp.float32),
        [pltpu.SemaphoreType.DMA for _ in range(2)],
        [pltpu.SemaphoreType.DMA for _ in range(2)],
    ],
    compiler_params=pltpu.CompilerParams(
        use_tc_tiling_on_sc=True, needs_layout_passes=False),
)
def _transpose_kernel(tT_hbm, rm_hbm, inb, outb, tailb, gsem, wsem):
    wid = lax.axis_index("s") * _NC + lax.axis_index("c")
    nblk = _NCK // _NW + jnp.where(wid < _NCK % _NW, 1, 0)

    d_lo = lax.iota(jnp.int32, 16)
    d_hi = d_lo + 16

    def load_blk(vt, p):
        off = pl.multiple_of(vt * _CHV, 128)
        return pltpu.async_copy(
            tT_hbm.at[:, pl.ds(off, _CHV)],
            inb[p].at[:, pl.ds(0, _CHV)], gsem[p])

    def wait_load(p):
        pltpu.make_async_copy(
            tT_hbm.at[:, pl.ds(0, _CHV)],
            inb[p].at[:, pl.ds(0, _CHV)], gsem[p]).wait()

    def store_blk(r0, p):
        return pltpu.async_copy(
            outb[p],
            rm_hbm.at[pl.ds(pl.multiple_of(r0, _RM_PER_CK), _RM_PER_CK)],
            wsem[p])

    def wait_store(p):
        pltpu.make_async_copy(
            outb[p], rm_hbm.at[pl.ds(0, _RM_PER_CK)], wsem[p]).wait()

    def permute(p):
        @plsc.parallel_loop(0, _RM_PER_CK, unroll=8)
        def _(j):
            for h in range(8):
                c = jnp.full((16,), 4 * j + h // 2, jnp.int32)
                d = d_lo if h % 2 == 0 else d_hi
                v = plsc.load_gather(inb[p], [d, c])
                outb[p][j, pl.ds(16 * h, 16)] = v

    def blk_index(i):
        return i * _NW + wid

    load_blk(blk_index(0), 0)

    @pl.loop(0, nblk)
    def _(i):
        p = (i % 2).astype(jnp.int32)
        for par in range(2):
            @pl.when(p == par)
            def _():
                @pl.when(i + 1 < nblk)
                def _():
                    load_blk(blk_index(i + 1), 1 - par)
                wait_load(par)
                @pl.when(i >= 2)
                def _():
                    wait_store(par)
                permute(par)
                store_blk(blk_index(i) * _RM_PER_CK, par)

    @pl.loop(0, jnp.minimum(nblk, 2))
    def _(i):
        for par in range(2):
            @pl.when(((nblk - 1 - i) % 2) == par)
            def _():
                wait_store(par)

    @pl.when(wid == _NW - 1)
    def _():
        pltpu.async_copy(
            tT_hbm.at[:, pl.ds(_NCK * _CHV, 64)], tailb, gsem[0]).wait()
        @pl.loop(0, 16)
        def _(j):
            for h in range(8):
                c = jnp.full((16,), 4 * j + h // 2, jnp.int32)
                d = d_lo if h % 2 == 0 else d_hi
                v = plsc.load_gather(tailb, [d, c])
                outb[0][j, pl.ds(16 * h, 16)] = v
        pltpu.async_copy(
            outb[0].at[pl.ds(0, 16)],
            rm_hbm.at[pl.ds(_RM_ROWS - 16, 16)], wsem[0]).wait()


@functools.partial(
    pl.kernel,
    out_type=jax.ShapeDtypeStruct((_B_TOTAL, _D), jnp.float32),
    mesh=_mesh,
    scratch_types=[
        pltpu.VMEM((_B_PER_W,), jnp.int32),
        [pltpu.VMEM((_CH, _D), jnp.float32) for _ in range(_NBUF)],
        [pltpu.SemaphoreType.DMA for _ in range(_NBUF)],
        [pltpu.SemaphoreType.DMA for _ in range(_NBUF)],
    ],
    compiler_params=pltpu.CompilerParams(use_tc_tiling_on_sc=False),
)
def _gather_kernel(idx_hbm, table_hbm, out_hbm, idx_v, rows, gsem, wsem):
    wid = lax.axis_index("s") * _NC + lax.axis_index("c")
    base = wid * _B_PER_W

    pltpu.sync_copy(idx_hbm.at[pl.ds(base, _B_PER_W)], idx_v)

    def gather(c, b):
        return pltpu.async_copy(
            table_hbm.at[idx_v.at[pl.ds(c * _CH, _CH)]], rows[b], gsem[b])

    def gather_wait(b):
        pltpu.make_async_copy(
            table_hbm.at[idx_v.at[pl.ds(0, _CH)]], rows[b], gsem[b]).wait()

    def write(c, b):
        return pltpu.async_copy(
            rows[b], out_hbm.at[pl.ds(base + c * _CH, _CH)], wsem[b])

    def write_wait(b):
        pltpu.make_async_copy(
            rows[b], out_hbm.at[pl.ds(base, _CH)], wsem[b]).wait()

    for b in range(_NBUF):
        gather(b, b)

    @pl.loop(0, _NCH, step=_NBUF)
    def _(c):
        for b in range(_NBUF):
            g = c + b
            gather_wait(b)
            write(g, b)

            @pl.when(g + _NBUF < _NCH)
            def _():
                write_wait(b)
                gather(g + _NBUF, b)

    for b in range(_NBUF):
        write_wait(b)


def kernel(x, table):
    table_rm = _transpose_kernel(table.T)
    out = _gather_kernel(x.reshape(-1), table_rm.reshape(_VOCAB, _D))
    return out.reshape(x.shape[0], x.shape[1], _D)
